# baseline (speedup 1.0000x reference)
import jax
import jax.numpy as jnp
import numpy as np
from jax.experimental import pallas as pl

_B = 1024
_D = 64
_C = 16
_K = 25
_MIN_CONF = 0.25
_BIG = 3.0e38
_BIG_BITS = int(np.float32(_BIG).view(np.int32))
_BISECT_ROUNDS = 22


def _overlap_body(enc_ref, cat_ref, ent_ref, ncomp_ref):
    enc = enc_ref[...]
    cat = cat_ref[...]

    sq = jnp.sum(enc * enc, axis=1)
    g = jnp.dot(enc, enc.T, preferred_element_type=jnp.float32)
    d2 = jnp.maximum(sq[:, None] + sq[None, :] - 2.0 * g, 0.0)

    bits = d2.view(jnp.int32)

    def bisect(_, carry):
        lo, hi = carry
        mid = lo + jax.lax.shift_right_logical(hi - lo + 1, 1)
        cnt = jnp.sum((bits < mid[:, None]).astype(jnp.int32), axis=1)
        ok = cnt <= _K
        return jnp.where(ok, mid, lo), jnp.where(ok, hi, mid - 1)

    hi0 = jnp.full((_B,), _BIG_BITS, jnp.int32)
    lo, _ = jax.lax.fori_loop(
        0, _BISECT_ROUNDS, bisect, (jnp.zeros((_B,), jnp.int32), hi0)
    )
    m0 = lo.view(jnp.float32)

    def walk_cond(carry):
        return carry[2]

    def walk_body(carry):
        m, thr, _ = carry
        gt = d2 > m[:, None]
        mn = jnp.min(jnp.where(gt, d2, _BIG), axis=1)
        n_above = jnp.sum(gt.astype(jnp.float32), axis=1)
        live = float(_B) - n_above <= float(_K)
        thr = jnp.where(live, mn, thr)
        return mn, thr, jnp.any(live)

    _, thresh2, _ = jax.lax.while_loop(
        walk_cond, walk_body, (m0, m0, jnp.bool_(True))
    )

    thresh = jnp.sqrt(thresh2)
    dist = jnp.sqrt(d2)
    mask = (dist < thresh[:, None]).astype(jnp.float32)
    counts = jnp.sum(mask, axis=1)

    cidx = jax.lax.broadcasted_iota(jnp.int32, (_B, _C), 1)
    maxg = jnp.max(cat, axis=1)
    hard = jnp.min(jnp.where(cat == maxg[:, None], cidx, _C), axis=1)
    onehot = (cidx == hard[:, None]).astype(jnp.float32)

    bins = jnp.dot(mask, onehot, preferred_element_type=jnp.float32)
    bins = bins / counts[:, None]
    ent = -jnp.sum(bins * jnp.log(bins + 1e-5), axis=1)
    ent_ref[...] = ent[:, None]

    conf = (maxg >= _MIN_CONF).astype(jnp.float32)
    populated = jnp.sum(onehot * conf[:, None], axis=0)
    ncomp_ref[...] = jnp.sum((populated > 0.0).astype(jnp.float32)).reshape(1, 1)


def kernel(encodings, categorical):
    ent, ncomp = pl.pallas_call(
        _overlap_body,
        out_shape=[
            jax.ShapeDtypeStruct((_B, 1), jnp.float32),
            jax.ShapeDtypeStruct((1, 1), jnp.float32),
        ],
    )(encodings, categorical)
    return encodings, ent.reshape(_B), ncomp.reshape(())


# speedup vs baseline: 1.0632x; 1.0632x over previous
import jax
import jax.numpy as jnp
import numpy as np
from jax.experimental import pallas as pl

_B = 1024
_D = 64
_C = 16
_K = 25
_MIN_CONF = 0.25
_BIG = 3.0e38
_BIG_BITS = int(np.float32(_BIG).view(np.int32))
_BISECT_ROUNDS = 16


def _overlap_body(enc_ref, cat_ref, ent_ref, ncomp_ref):
    enc = enc_ref[...]
    cat = cat_ref[...]

    sq = jnp.sum(enc * enc, axis=1)
    g = jnp.dot(enc, enc.T, preferred_element_type=jnp.float32)
    d2 = jnp.maximum(sq[:, None] + sq[None, :] - 2.0 * g, 0.0)

    bits = d2.view(jnp.int32)

    def bisect(_, carry):
        lo, hi = carry
        mid = lo + jax.lax.shift_right_logical(hi - lo + 1, 1)
        cnt = jnp.sum((bits < mid[:, None]).astype(jnp.int32), axis=1)
        ok = cnt <= _K
        return jnp.where(ok, mid, lo), jnp.where(ok, hi, mid - 1)

    hi0 = jnp.full((_B,), _BIG_BITS, jnp.int32)
    lo, _ = jax.lax.fori_loop(
        0, _BISECT_ROUNDS, bisect, (jnp.zeros((_B,), jnp.int32), hi0)
    )
    m0 = lo.view(jnp.float32)

    def walk_cond(carry):
        return carry[2]

    def walk_body(carry):
        m, thr, _ = carry
        gt = d2 > m[:, None]
        mn = jnp.min(jnp.where(gt, d2, _BIG), axis=1)
        n_above = jnp.sum(gt.astype(jnp.float32), axis=1)
        live = float(_B) - n_above <= float(_K)
        thr = jnp.where(live, mn, thr)
        return mn, thr, jnp.any(live)

    _, thresh2, _ = jax.lax.while_loop(
        walk_cond, walk_body, (m0, m0, jnp.bool_(True))
    )

    thresh = jnp.sqrt(thresh2)
    dist = jnp.sqrt(d2)
    mask = (dist < thresh[:, None]).astype(jnp.float32)
    counts = jnp.sum(mask, axis=1)

    cidx = jax.lax.broadcasted_iota(jnp.int32, (_B, _C), 1)
    maxg = jnp.max(cat, axis=1)
    hard = jnp.min(jnp.where(cat == maxg[:, None], cidx, _C), axis=1)
    onehot = (cidx == hard[:, None]).astype(jnp.float32)

    bins = jnp.dot(mask, onehot, preferred_element_type=jnp.float32)
    bins = bins / counts[:, None]
    ent = -jnp.sum(bins * jnp.log(bins + 1e-5), axis=1)
    ent_ref[...] = ent[:, None]

    conf = (maxg >= _MIN_CONF).astype(jnp.float32)
    populated = jnp.sum(onehot * conf[:, None], axis=0)
    ncomp_ref[...] = jnp.sum((populated > 0.0).astype(jnp.float32)).reshape(1, 1)


def kernel(encodings, categorical):
    ent, ncomp = pl.pallas_call(
        _overlap_body,
        out_shape=[
            jax.ShapeDtypeStruct((_B, 1), jnp.float32),
            jax.ShapeDtypeStruct((1, 1), jnp.float32),
        ],
    )(encodings, categorical)
    return encodings, ent.reshape(_B), ncomp.reshape(())


# speedup vs baseline: 1.0712x; 1.0075x over previous
import jax
import jax.numpy as jnp
import numpy as np
from jax.experimental import pallas as pl

_B = 1024
_D = 64
_C = 16
_K = 25
_MIN_CONF = 0.25
_BIG = 3.0e38
_BIG_BITS = int(np.float32(_BIG).view(np.int32))
_BISECT_ROUNDS = 17


def _overlap_body(enc_ref, cat_ref, ent_ref, ncomp_ref):
    enc = enc_ref[...]
    cat = cat_ref[...]

    sq = jnp.sum(enc * enc, axis=1)
    g = jnp.dot(enc, enc.T, preferred_element_type=jnp.float32)
    d2 = jnp.maximum(sq[:, None] + sq[None, :] - 2.0 * g, 0.0)

    bits = d2.view(jnp.int32)

    def bisect(_, carry):
        lo, hi = carry
        mid = lo + jax.lax.shift_right_logical(hi - lo + 1, 1)
        cnt = jnp.sum((bits < mid[:, None]).astype(jnp.int32), axis=1)
        ok = cnt <= _K
        return jnp.where(ok, mid, lo), jnp.where(ok, hi, mid - 1)

    hi0 = jnp.full((_B,), _BIG_BITS, jnp.int32)
    lo, _ = jax.lax.fori_loop(
        0, _BISECT_ROUNDS, bisect, (jnp.zeros((_B,), jnp.int32), hi0)
    )
    m0 = lo.view(jnp.float32)

    def walk_cond(carry):
        return carry[2]

    def walk_body(carry):
        m, thr, _ = carry
        gt = d2 > m[:, None]
        mn = jnp.min(jnp.where(gt, d2, _BIG), axis=1)
        n_above = jnp.sum(gt.astype(jnp.float32), axis=1)
        live = float(_B) - n_above <= float(_K)
        thr = jnp.where(live, mn, thr)
        return mn, thr, jnp.any(live)

    _, thresh2, _ = jax.lax.while_loop(
        walk_cond, walk_body, (m0, m0, jnp.bool_(True))
    )

    thresh = jnp.sqrt(thresh2)
    dist = jnp.sqrt(d2)
    mask = (dist < thresh[:, None]).astype(jnp.float32)
    counts = jnp.sum(mask, axis=1)

    cidx = jax.lax.broadcasted_iota(jnp.int32, (_B, _C), 1)
    maxg = jnp.max(cat, axis=1)
    hard = jnp.min(jnp.where(cat == maxg[:, None], cidx, _C), axis=1)
    onehot = (cidx == hard[:, None]).astype(jnp.float32)

    bins = jnp.dot(mask, onehot, preferred_element_type=jnp.float32)
    bins = bins / counts[:, None]
    ent = -jnp.sum(bins * jnp.log(bins + 1e-5), axis=1)
    ent_ref[...] = ent[:, None]

    conf = (maxg >= _MIN_CONF).astype(jnp.float32)
    populated = jnp.sum(onehot * conf[:, None], axis=0)
    ncomp_ref[...] = jnp.sum((populated > 0.0).astype(jnp.float32)).reshape(1, 1)


def kernel(encodings, categorical):
    ent, ncomp = pl.pallas_call(
        _overlap_body,
        out_shape=[
            jax.ShapeDtypeStruct((_B, 1), jnp.float32),
            jax.ShapeDtypeStruct((1, 1), jnp.float32),
        ],
    )(encodings, categorical)
    return encodings, ent.reshape(_B), ncomp.reshape(())
